# Initial kernel scaffold; baseline (speedup 1.0000x reference)
import jax
import jax.numpy as jnp
from jax.experimental import pallas as pl


def kernel(x, edge_index, edge_weight, W1, b1, W2, b2):
    raise NotImplementedError("write your pallas kernel here")


# speedup vs baseline: 12.2042x; 12.2042x over previous
import jax
import jax.numpy as jnp
from jax import lax
from jax.experimental import pallas as pl
from jax.experimental.pallas import tpu as pltpu
from jax.experimental.pallas import tpu_sc as plsc

N = 10000
E = 320000
D = 128
H = 128
C = 40
CP = 64

NC = 2
NS = 16
NW = NC * NS
EPT = E // NW
K = 80
CPT = E // NS // K
NP = 10240
SPT = NP // NS

_MESH = plsc.VectorSubcoreMesh(core_axis_name="c", subcore_axis_name="s")
_SC_PARAMS = pltpu.CompilerParams(
    needs_layout_passes=False, use_tc_tiling_on_sc=False
)


def _deg_body(dst_hbm, ew_hbm, out_hbm, dstv, ewv, degv):
    cid = lax.axis_index("c")
    sid = lax.axis_index("s")
    wid = cid * NS + sid
    pltpu.sync_copy(dst_hbm.at[wid], dstv)
    pltpu.sync_copy(ew_hbm.at[wid], ewv)

    zero16 = jnp.zeros((16,), jnp.float32)

    def zbody(i, carry):
        degv[pl.ds(i * 16, 16)] = zero16
        return carry

    lax.fori_loop(0, N // 16, zbody, 0)

    def abody(i, carry):
        idx = dstv[pl.ds(i * 16, 16)]
        w = ewv[pl.ds(i * 16, 16)]
        plsc.addupdate_scatter(degv, [idx], w)
        return carry

    lax.fori_loop(0, EPT // 16, abody, 0)
    pltpu.sync_copy(degv, out_hbm.at[wid])


_deg_call = pl.kernel(
    _deg_body,
    out_type=jax.ShapeDtypeStruct((NW, N), jnp.float32),
    mesh=_MESH,
    scratch_types=[
        pltpu.VMEM((EPT,), jnp.int32),
        pltpu.VMEM((EPT,), jnp.float32),
        pltpu.VMEM((N,), jnp.float32),
    ],
    compiler_params=_SC_PARAMS,
)


def _make_agg(width):
    ngrp = width // 16

    def body(h_hbm, src_hbm, dst_hbm, ew_hbm, z_hbm, out_hbm,
             srcv, dstv, ewv, rows_a, rows_b, acc, sem_a, sem_b):
        cid = lax.axis_index("c")
        sid = lax.axis_index("s")
        pltpu.sync_copy(src_hbm.at[sid], srcv)
        pltpu.sync_copy(dst_hbm.at[sid], dstv)
        pltpu.sync_copy(ew_hbm.at[sid], ewv)
        sbase = pl.multiple_of(sid * SPT, 8)
        pltpu.sync_copy(z_hbm, acc.at[pl.ds(sbase, SPT)])
        plsc.subcore_barrier()

        hc = h_hbm.at[cid]

        def process(rows_ref, j):
            def sgrp(eg, carry):
                cfv = ewv[j, pl.ds(eg * 16, 16)]
                for l in range(16):
                    cf = cfv[l]
                    for g in range(ngrp):
                        sl = pl.ds(g * 16, 16)
                        rows_ref[eg * 16 + l, sl] = rows_ref[eg * 16 + l, sl] * cf
                return carry

            lax.fori_loop(0, K // 16, sgrp, 0)
            pltpu.sync_copy(rows_ref, acc.at[dstv.at[j]], add=True)

        def lbody(t, carry):
            j0 = 2 * t
            pltpu.async_copy(hc.at[srcv.at[j0]], rows_a, sem_a)
            pltpu.async_copy(hc.at[srcv.at[j0 + 1]], rows_b, sem_b)
            pltpu.make_async_copy(hc.at[srcv.at[j0]], rows_a, sem_a).wait()
            process(rows_a, j0)
            pltpu.make_async_copy(hc.at[srcv.at[j0 + 1]], rows_b, sem_b).wait()
            process(rows_b, j0 + 1)
            return carry

        lax.fori_loop(0, CPT // 2, lbody, 0)
        plsc.subcore_barrier()
        pltpu.sync_copy(acc.at[pl.ds(sbase, SPT)],
                        out_hbm.at[cid, pl.ds(sbase, SPT)])

    return pl.kernel(
        body,
        out_type=jax.ShapeDtypeStruct((NC, NP, width), jnp.float32),
        mesh=_MESH,
        scratch_types=[
            pltpu.VMEM((CPT, K), jnp.int32),
            pltpu.VMEM((CPT, K), jnp.int32),
            pltpu.VMEM((CPT, K), jnp.float32),
            pltpu.VMEM((K, width), jnp.float32),
            pltpu.VMEM((K, width), jnp.float32),
            pltpu.VMEM_SHARED((NP, width), jnp.float32),
            pltpu.SemaphoreType.DMA,
            pltpu.SemaphoreType.DMA,
        ],
        compiler_params=_SC_PARAMS,
    )


_agg_h = _make_agg(H // NC)
_agg_c = _make_agg(CP // NC)


def _tc_pre_body(degp_ref, x_ref, w1_ref, hs_ref, h1_ref, dinv_ref):
    ones = jnp.ones((NW, 1), jnp.float32)
    deg = lax.dot_general(degp_ref[...], ones, (((0,), (0,)), ((), ()))) + 1.0
    dinv = jnp.where(deg > 0, lax.rsqrt(jnp.maximum(deg, 1e-12)), 0.0)
    h = jnp.dot(x_ref[...], w1_ref[...], preferred_element_type=jnp.float32)
    h = h * dinv
    h1_ref[...] = h
    hs_ref[0] = h[:, : H // NC]
    hs_ref[1] = h[:, H // NC:]
    dinv_ref[...] = dinv


def _tc_mid_body(p_ref, h1_ref, dinv_ref, b1_ref, w2_ref, ys_ref, y_ref):
    agg = jnp.concatenate([p_ref[0, :N], p_ref[1, :N]], axis=1)
    s = (agg + h1_ref[...]) * dinv_ref[...] + b1_ref[...]
    z = jnp.maximum(s, 0.0)
    y = jnp.dot(z, w2_ref[...], preferred_element_type=jnp.float32)
    y = y * dinv_ref[...]
    y_ref[...] = y
    ys_ref[0] = y[:, : CP // NC]
    ys_ref[1] = y[:, CP // NC:]


def _tc_post_body(q_ref, y_ref, dinv_ref, b2_ref, o_ref):
    agg = jnp.concatenate([q_ref[0, :N], q_ref[1, :N]], axis=1)
    s = (agg + y_ref[...]) * dinv_ref[...]
    t = s[:, :C] + b2_ref[...]
    m = jnp.max(t, axis=1, keepdims=True)
    u = t - m
    lse = jnp.log(jnp.sum(jnp.exp(u), axis=1, keepdims=True))
    o_ref[...] = u - lse


_tc_pre = pl.pallas_call(
    _tc_pre_body,
    out_shape=[
        jax.ShapeDtypeStruct((NC, N, H // NC), jnp.float32),
        jax.ShapeDtypeStruct((N, H), jnp.float32),
        jax.ShapeDtypeStruct((N, 1), jnp.float32),
    ],
)

_tc_mid = pl.pallas_call(
    _tc_mid_body,
    out_shape=[
        jax.ShapeDtypeStruct((NC, N, CP // NC), jnp.float32),
        jax.ShapeDtypeStruct((N, CP), jnp.float32),
    ],
)

_tc_post = pl.pallas_call(
    _tc_post_body,
    out_shape=jax.ShapeDtypeStruct((N, C), jnp.float32),
)


@jax.jit
def kernel(x, edge_index, edge_weight, W1, b1, W2, b2):
    src = edge_index[0]
    dst = edge_index[1]
    src3d = src.reshape(NS, CPT, K)
    dst3d = dst.reshape(NS, CPT, K)
    ew3d = edge_weight.reshape(NS, CPT, K)
    dst_t = dst.reshape(NW, EPT)
    ew_t = edge_weight.reshape(NW, EPT)
    w2p = jnp.zeros((H, CP), jnp.float32).at[:, :C].set(W2)
    b1r = b1.reshape(1, H)
    b2r = b2.reshape(1, C)
    zh = jnp.zeros((SPT, H // NC), jnp.float32)
    zc = jnp.zeros((SPT, CP // NC), jnp.float32)

    degp = _deg_call(dst_t, ew_t)
    hs, h1, dinv = _tc_pre(degp, x, W1)
    p = _agg_h(hs, src3d, dst3d, ew3d, zh)
    ys, y = _tc_mid(p, h1, dinv, b1r, w2p)
    q = _agg_c(ys, src3d, dst3d, ew3d, zc)
    return _tc_post(q, y, dinv, b2r)
